# baseline (speedup 1.0000x reference)
import functools

import jax
import jax.numpy as jnp
from jax import lax
from jax.experimental import pallas as pl
from jax.experimental.pallas import tpu as pltpu
from jax.experimental.pallas import tpu_sc as plsc

_MIN_RATING = 0.0
_SCALE = 100.0
_NUM_BINS = 1001
_BATCH = 16384
_NS, _L = 16, 16
_BPW = _BATCH // _NS
_VECS = _BPW // _L


def _body(pred_hbm, targ_hbm, w_hbm, out_hbm,
          w_v, pred_v, targ_v, part_v, red_v, shared, sem):
    sid = lax.axis_index("s")
    base = sid * _BPW
    cw = pltpu.async_copy(w_hbm, w_v, sem)
    cp = pltpu.async_copy(pred_hbm.at[pl.ds(base, _BPW)], pred_v, sem)
    ct = pltpu.async_copy(targ_hbm.at[pl.ds(base, _BPW)], targ_v, sem)
    cw.wait()
    cp.wait()
    ct.wait()

    @plsc.parallel_loop(0, _BPW, step=_L, unroll=8,
                        carry=jnp.zeros((_L,), jnp.float32))
    def _acc(off, acc):
        p = pred_v[pl.ds(off, _L)]
        t = targ_v[pl.ds(off, _L)]
        idx = ((t - _MIN_RATING) * _SCALE + 0.5).astype(jnp.int32)
        idx = jnp.minimum(jnp.maximum(idx, 0), _NUM_BINS - 1)
        w = plsc.load_gather(w_v, [idx])
        d = p - t
        return acc + w * d * d
    acc = _acc

    psum = lax.reduce_sum_p.bind(acc, axes=(0,))
    part_v[...] = jnp.full((_L,), 0.0, jnp.float32) + psum
    pltpu.sync_copy(part_v, shared.at[pl.ds(sid * _L, _L)])
    plsc.subcore_barrier()

    @pl.when(sid == 0)
    def _():
        pltpu.sync_copy(shared, red_v)
        stride_idx = lax.iota(jnp.int32, _L) * _L
        tile_sums = plsc.load_gather(red_v, [stride_idx])
        mean = lax.reduce_sum_p.bind(tile_sums * (1.0 / _BATCH), axes=(0,))
        part_v[...] = jnp.full((_L,), 0.0, jnp.float32) + mean
        pltpu.sync_copy(part_v, out_hbm)


@functools.partial(jax.jit, static_argnames=())
def kernel(predictions, targets, weight_tensor):
    mesh = plsc.VectorSubcoreMesh(
        core_axis_name="c", subcore_axis_name="s", num_cores=1)
    out = pl.kernel(
        _body,
        out_type=jax.ShapeDtypeStruct((_L,), jnp.float32),
        mesh=mesh,
        scratch_types=[
            pltpu.VMEM((_NUM_BINS,), jnp.float32),
            pltpu.VMEM((_BPW,), jnp.float32),
            pltpu.VMEM((_BPW,), jnp.float32),
            pltpu.VMEM((_L,), jnp.float32),
            pltpu.VMEM((_NS * _L,), jnp.float32),
            pltpu.VMEM_SHARED((_NS * _L,), jnp.float32),
            pltpu.SemaphoreType.DMA,
        ],
        compiler_params=pltpu.CompilerParams(needs_layout_passes=False),
    )(predictions, targets, weight_tensor)
    return out[0]


# speedup vs baseline: 1.0037x; 1.0037x over previous
import functools

import jax
import jax.numpy as jnp
from jax import lax
from jax.experimental import pallas as pl
from jax.experimental.pallas import tpu as pltpu
from jax.experimental.pallas import tpu_sc as plsc

_MIN_RATING = 0.0
_SCALE = 100.0
_NUM_BINS = 1001
_BATCH = 16384
_NS, _L = 16, 16
_BPW = _BATCH // _NS
_VECS = _BPW // _L


def _body(pred_hbm, targ_hbm, w_hbm, out_hbm,
          w_v, pred_v, targ_v, part_v, red_v, shared, sem):
    sid = lax.axis_index("s")
    base = sid * _BPW
    cw = pltpu.async_copy(w_hbm, w_v, sem)
    cp = pltpu.async_copy(pred_hbm.at[pl.ds(base, _BPW)], pred_v, sem)
    ct = pltpu.async_copy(targ_hbm.at[pl.ds(base, _BPW)], targ_v, sem)
    cw.wait()
    cp.wait()
    ct.wait()

    @plsc.parallel_loop(0, _BPW, step=_L, unroll=4,
                        carry=jnp.zeros((_L,), jnp.float32))
    def _acc(off, acc):
        p = pred_v[pl.ds(off, _L)]
        t = targ_v[pl.ds(off, _L)]
        idx = ((t - _MIN_RATING) * _SCALE + 0.5).astype(jnp.int32)
        idx = jnp.minimum(jnp.maximum(idx, 0), _NUM_BINS - 1)
        w = plsc.load_gather(w_v, [idx])
        d = p - t
        return acc + w * d * d
    acc = _acc

    psum = lax.reduce_sum_p.bind(acc, axes=(0,))
    part_v[...] = jnp.full((_L,), 0.0, jnp.float32) + psum
    pltpu.sync_copy(part_v, shared.at[pl.ds(sid * _L, _L)])
    plsc.subcore_barrier()

    @pl.when(sid == 0)
    def _():
        pltpu.sync_copy(shared, red_v)
        stride_idx = lax.iota(jnp.int32, _L) * _L
        tile_sums = plsc.load_gather(red_v, [stride_idx])
        mean = lax.reduce_sum_p.bind(tile_sums * (1.0 / _BATCH), axes=(0,))
        part_v[...] = jnp.full((_L,), 0.0, jnp.float32) + mean
        pltpu.sync_copy(part_v, out_hbm)


@functools.partial(jax.jit, static_argnames=())
def kernel(predictions, targets, weight_tensor):
    mesh = plsc.VectorSubcoreMesh(
        core_axis_name="c", subcore_axis_name="s", num_cores=1)
    out = pl.kernel(
        _body,
        out_type=jax.ShapeDtypeStruct((_L,), jnp.float32),
        mesh=mesh,
        scratch_types=[
            pltpu.VMEM((_NUM_BINS,), jnp.float32),
            pltpu.VMEM((_BPW,), jnp.float32),
            pltpu.VMEM((_BPW,), jnp.float32),
            pltpu.VMEM((_L,), jnp.float32),
            pltpu.VMEM((_NS * _L,), jnp.float32),
            pltpu.VMEM_SHARED((_NS * _L,), jnp.float32),
            pltpu.SemaphoreType.DMA,
        ],
        compiler_params=pltpu.CompilerParams(needs_layout_passes=False),
    )(predictions, targets, weight_tensor)
    return out[0]
